# baseline (speedup 1.0000x reference)
import functools

import jax
import jax.numpy as jnp
from jax.experimental import pallas as pl
from jax.experimental.pallas import tpu as pltpu


def _conv_pool_stats_kernel(a_ref, b_ref, w_ref, out_ref, stats_ref, *, TH, W, C):
    THp, Wp = TH // 2, W // 2
    x = jnp.concatenate([a_ref[0], b_ref[0]], axis=0)
    Cin = x.shape[-1]

    xc = jnp.concatenate(
        [x[:, 0:W, :], x[:, 1:W + 1, :], x[:, 2:W + 2, :]], axis=-1)

    a = jnp.dot(xc.reshape((TH + 2) * W, 3 * Cin), w_ref[...],
                preferred_element_type=jnp.float32)
    a = a.reshape(TH + 2, W, 3 * C)

    conv = (a[0:TH, :, 0:C]
            + a[1:TH + 1, :, C:2 * C]
            + a[2:TH + 2, :, 2 * C:3 * C])

    c = conv.reshape(THp, 2, W, C)
    mh = jnp.maximum(c[:, 0], c[:, 1])
    mh2 = mh.reshape(THp, Wp, 2 * C)
    pooled = jnp.maximum(mh2[:, :, :C], mh2[:, :, C:])
    pooled = pooled.reshape(THp * Wp, C)
    out_ref[0] = pooled

    s = jnp.sum(pooled, axis=0, keepdims=True)
    ss = jnp.sum(pooled * pooled, axis=0, keepdims=True)
    stats_ref[0] = jnp.concatenate([s, ss], axis=0)


def _bn_relu_kernel(x_ref, scale_ref, shift_ref, o_ref):
    y = jnp.maximum(x_ref[0] * scale_ref[...] + shift_ref[...], 0.0)
    o_ref[0] = y.T


def kernel(x_nchw, w, b, gamma, beta):
    del b
    eps = 1e-5
    N, Cin, H, W = x_nchw.shape
    Cout = w.shape[0]
    assert H % 2 == 0 and W % 2 == 0
    Hp, Wp = H // 2, W // 2

    TH = 32
    while H % TH != 0:
        TH //= 2
    THp = TH // 2
    nH = H // TH

    x = jnp.transpose(x_nchw, (0, 2, 3, 1)).astype(jnp.bfloat16)
    xp = jnp.pad(x, ((0, 0), (1, 1), (1, 1), (0, 0)))
    wN = jnp.transpose(w, (3, 1, 2, 0)).reshape(3 * Cin, 3 * Cout)
    wN = wN.astype(jnp.bfloat16)

    cparams = pltpu.CompilerParams(
        dimension_semantics=("parallel", "parallel"),
        vmem_limit_bytes=100 * 1024 * 1024,
    )

    k1 = functools.partial(_conv_pool_stats_kernel, TH=TH, W=W, C=Cout)
    pooled, stats = pl.pallas_call(
        k1,
        grid=(N, nH),
        in_specs=[
            pl.BlockSpec((1, TH, W + 2, Cin), lambda n, h: (n, h, 0, 0)),
            pl.BlockSpec((1, 2, W + 2, Cin), lambda n, h: (n, THp * (h + 1), 0, 0)),
            pl.BlockSpec((3 * Cin, 3 * Cout), lambda n, h: (0, 0)),
        ],
        out_specs=[
            pl.BlockSpec((1, THp * Wp, Cout), lambda n, h: (n, h, 0)),
            pl.BlockSpec((1, 2, Cout), lambda n, h: (n * nH + h, 0, 0)),
        ],
        out_shape=[
            jax.ShapeDtypeStruct((N, Hp * Wp, Cout), jnp.float32),
            jax.ShapeDtypeStruct((N * nH, 2, Cout), jnp.float32),
        ],
        compiler_params=cparams,
    )(xp, xp, wN)

    M2 = N * Hp * Wp
    ssum = jnp.sum(stats, axis=0)
    mean = ssum[0] / M2
    var = jnp.maximum(ssum[1] / M2 - mean * mean, 0.0)
    scale = gamma.astype(jnp.float32) * jax.lax.rsqrt(var + eps)
    shift = beta.astype(jnp.float32) - mean * scale
    scale2 = scale.reshape(1, Cout)
    shift2 = shift.reshape(1, Cout)

    S = Hp * Wp
    TR = 2048
    while S % TR != 0:
        TR //= 2
    out3d = pl.pallas_call(
        _bn_relu_kernel,
        grid=(N, S // TR),
        in_specs=[
            pl.BlockSpec((1, TR, Cout), lambda n, i: (n, i, 0)),
            pl.BlockSpec((1, Cout), lambda n, i: (0, 0)),
            pl.BlockSpec((1, Cout), lambda n, i: (0, 0)),
        ],
        out_specs=pl.BlockSpec((1, Cout, TR), lambda n, i: (n, 0, i)),
        out_shape=jax.ShapeDtypeStruct((N, Cout, S), jnp.float32),
        compiler_params=pltpu.CompilerParams(
            dimension_semantics=("parallel", "parallel"),
            vmem_limit_bytes=64 * 1024 * 1024,
        ),
    )(pooled, scale2, shift2)

    return out3d.reshape(N, Cout, Hp, Wp)


# speedup vs baseline: 1.1632x; 1.1632x over previous
import functools

import jax
import jax.numpy as jnp
from jax.experimental import pallas as pl
from jax.experimental.pallas import tpu as pltpu


def _conv_pool_stats_kernel(a_ref, b_ref, w_ref, out_ref, stats_ref, *, TH, W, C):
    THp, Wp = TH // 2, W // 2
    x = jnp.concatenate([a_ref[0], b_ref[0]], axis=0)
    Cin = x.shape[-1]

    xc = jnp.concatenate(
        [x[:, 0:W, :], x[:, 1:W + 1, :], x[:, 2:W + 2, :]], axis=-1)

    a = jnp.dot(xc.reshape((TH + 2) * W, 3 * Cin), w_ref[...],
                preferred_element_type=jnp.float32)
    a = a.reshape(TH + 2, W, 3 * C)

    conv = (a[0:TH, :, 0:C]
            + a[1:TH + 1, :, C:2 * C]
            + a[2:TH + 2, :, 2 * C:3 * C])

    c = conv.reshape(THp, 2, W, C)
    mh = jnp.maximum(c[:, 0], c[:, 1])
    mh2 = mh.reshape(THp, Wp, 2 * C)
    pooled = jnp.maximum(mh2[:, :, :C], mh2[:, :, C:])
    pooled = pooled.reshape(THp * Wp, C)
    out_ref[0] = pooled

    s = jnp.sum(pooled, axis=0, keepdims=True)
    ss = jnp.sum(pooled * pooled, axis=0, keepdims=True)
    stats_ref[0] = jnp.concatenate([s, ss], axis=0)


def _bn_relu_kernel(x_ref, scale_ref, shift_ref, o_ref):
    o_ref[...] = jnp.maximum(x_ref[...] * scale_ref[...] + shift_ref[...], 0.0)


def kernel(x_nchw, w, b, gamma, beta):
    del b
    eps = 1e-5
    N, Cin, H, W = x_nchw.shape
    Cout = w.shape[0]
    assert H % 2 == 0 and W % 2 == 0
    Hp, Wp = H // 2, W // 2

    TH = 32
    while H % TH != 0:
        TH //= 2
    THp = TH // 2
    nH = H // TH

    x = jnp.transpose(x_nchw.astype(jnp.bfloat16), (0, 2, 3, 1))
    xp = jnp.pad(x, ((0, 0), (1, 1), (1, 1), (0, 0)))
    wN = jnp.transpose(w, (3, 1, 2, 0)).reshape(3 * Cin, 3 * Cout)
    wN = wN.astype(jnp.bfloat16)

    cparams = pltpu.CompilerParams(
        dimension_semantics=("parallel", "parallel"),
        vmem_limit_bytes=100 * 1024 * 1024,
    )

    k1 = functools.partial(_conv_pool_stats_kernel, TH=TH, W=W, C=Cout)
    pooled, stats = pl.pallas_call(
        k1,
        grid=(N, nH),
        in_specs=[
            pl.BlockSpec((1, TH, W + 2, Cin), lambda n, h: (n, h, 0, 0)),
            pl.BlockSpec((1, 2, W + 2, Cin), lambda n, h: (n, THp * (h + 1), 0, 0)),
            pl.BlockSpec((3 * Cin, 3 * Cout), lambda n, h: (0, 0)),
        ],
        out_specs=[
            pl.BlockSpec((1, THp * Wp, Cout), lambda n, h: (n, h, 0)),
            pl.BlockSpec((1, 2, Cout), lambda n, h: (n * nH + h, 0, 0)),
        ],
        out_shape=[
            jax.ShapeDtypeStruct((N, Hp * Wp, Cout), jnp.float32),
            jax.ShapeDtypeStruct((N * nH, 2, Cout), jnp.float32),
        ],
        compiler_params=cparams,
    )(xp, xp, wN)

    M2 = N * Hp * Wp
    ssum = jnp.sum(stats, axis=0)
    mean = ssum[0] / M2
    var = jnp.maximum(ssum[1] / M2 - mean * mean, 0.0)
    scale = gamma.astype(jnp.float32) * jax.lax.rsqrt(var + eps)
    shift = beta.astype(jnp.float32) - mean * scale
    scale2 = scale.reshape(1, Cout)
    shift2 = shift.reshape(1, Cout)

    TR = 4096
    while M2 % TR != 0:
        TR //= 2
    pooled2d = pooled.reshape(M2, Cout)
    out2d = pl.pallas_call(
        _bn_relu_kernel,
        grid=(M2 // TR,),
        in_specs=[
            pl.BlockSpec((TR, Cout), lambda i: (i, 0)),
            pl.BlockSpec((1, Cout), lambda i: (0, 0)),
            pl.BlockSpec((1, Cout), lambda i: (0, 0)),
        ],
        out_specs=pl.BlockSpec((TR, Cout), lambda i: (i, 0)),
        out_shape=jax.ShapeDtypeStruct((M2, Cout), jnp.float32),
        compiler_params=pltpu.CompilerParams(
            dimension_semantics=("parallel",),
            vmem_limit_bytes=64 * 1024 * 1024,
        ),
    )(pooled2d, scale2, shift2)

    out = out2d.reshape(N, Hp, Wp, Cout)
    return jnp.transpose(out, (0, 3, 1, 2))


# speedup vs baseline: 1.3998x; 1.2034x over previous
import functools

import jax
import jax.numpy as jnp
from jax.experimental import pallas as pl
from jax.experimental.pallas import tpu as pltpu


def _conv_pool_stats_kernel(a_ref, t_ref, b_ref, w_ref, out_ref, stats_ref,
                            *, TH, W, C, nH):
    THp, Wp = TH // 2, W // 2
    h = pl.program_id(1)
    xm = a_ref[0]
    top = jnp.where(h == 0, 0.0, t_ref[0][:, 7:8, :])
    bot = jnp.where(h == nH - 1, 0.0, b_ref[0][:, 0:1, :])
    x3 = jnp.concatenate([top, xm, bot], axis=1)
    xt = jnp.transpose(x3.astype(jnp.bfloat16), (1, 2, 0))

    Cin = xt.shape[-1]
    z = jnp.zeros((TH + 2, 1, Cin), jnp.bfloat16)
    left = jnp.concatenate([z, xt[:, :-1, :]], axis=1)
    right = jnp.concatenate([xt[:, 1:, :], z], axis=1)
    xc = jnp.concatenate([left, xt, right], axis=-1)

    a = jnp.dot(xc.reshape((TH + 2) * W, 3 * Cin), w_ref[...],
                preferred_element_type=jnp.float32)
    a = a.reshape(TH + 2, W, 3 * C)

    conv = (a[0:TH, :, 0:C]
            + a[1:TH + 1, :, C:2 * C]
            + a[2:TH + 2, :, 2 * C:3 * C])

    c = conv.reshape(THp, 2, W, C)
    mh = jnp.maximum(c[:, 0], c[:, 1])
    mh2 = mh.reshape(THp, Wp, 2 * C)
    pooled = jnp.maximum(mh2[:, :, :C], mh2[:, :, C:])
    pooled = pooled.reshape(THp * Wp, C)
    out_ref[0] = pooled

    s = jnp.sum(pooled, axis=0, keepdims=True)
    ss = jnp.sum(pooled * pooled, axis=0, keepdims=True)
    stats_ref[0] = jnp.concatenate([s, ss], axis=0)


def _bn_relu_kernel(x_ref, scale_ref, shift_ref, o_ref):
    o_ref[...] = jnp.maximum(x_ref[...] * scale_ref[...] + shift_ref[...], 0.0)


def kernel(x_nchw, w, b, gamma, beta):
    del b
    eps = 1e-5
    N, Cin, H, W = x_nchw.shape
    Cout = w.shape[0]
    assert H % 2 == 0 and W % 2 == 0
    Hp, Wp = H // 2, W // 2

    TH = 32
    while H % TH != 0:
        TH //= 2
    assert TH % 8 == 0, "row tile must be a multiple of the 8-row halo blocks"
    THp = TH // 2
    nH = H // TH
    TH8 = TH // 8
    nR8 = H // 8

    wN = jnp.transpose(w, (3, 1, 2, 0)).reshape(3 * Cin, 3 * Cout)
    wN = wN.astype(jnp.bfloat16)

    cparams = pltpu.CompilerParams(
        dimension_semantics=("parallel", "parallel"),
        vmem_limit_bytes=100 * 1024 * 1024,
    )

    k1 = functools.partial(_conv_pool_stats_kernel, TH=TH, W=W, C=Cout, nH=nH)
    pooled, stats = pl.pallas_call(
        k1,
        grid=(N, nH),
        in_specs=[
            pl.BlockSpec((1, Cin, TH, W), lambda n, h: (n, 0, h, 0)),
            pl.BlockSpec((1, Cin, 8, W),
                         lambda n, h: (n, 0, jnp.maximum(h * TH8 - 1, 0), 0)),
            pl.BlockSpec((1, Cin, 8, W),
                         lambda n, h: (n, 0, jnp.minimum((h + 1) * TH8, nR8 - 1), 0)),
            pl.BlockSpec((3 * Cin, 3 * Cout), lambda n, h: (0, 0)),
        ],
        out_specs=[
            pl.BlockSpec((1, THp * Wp, Cout), lambda n, h: (n, h, 0)),
            pl.BlockSpec((1, 2, Cout), lambda n, h: (n * nH + h, 0, 0)),
        ],
        out_shape=[
            jax.ShapeDtypeStruct((N, Hp * Wp, Cout), jnp.float32),
            jax.ShapeDtypeStruct((N * nH, 2, Cout), jnp.float32),
        ],
        compiler_params=cparams,
    )(x_nchw, x_nchw, x_nchw, wN)

    M2 = N * Hp * Wp
    ssum = jnp.sum(stats, axis=0)
    mean = ssum[0] / M2
    var = jnp.maximum(ssum[1] / M2 - mean * mean, 0.0)
    scale = gamma.astype(jnp.float32) * jax.lax.rsqrt(var + eps)
    shift = beta.astype(jnp.float32) - mean * scale
    scale2 = scale.reshape(1, Cout)
    shift2 = shift.reshape(1, Cout)

    TR = 4096
    while M2 % TR != 0:
        TR //= 2
    pooled2d = pooled.reshape(M2, Cout)
    out2d = pl.pallas_call(
        _bn_relu_kernel,
        grid=(M2 // TR,),
        in_specs=[
            pl.BlockSpec((TR, Cout), lambda i: (i, 0)),
            pl.BlockSpec((1, Cout), lambda i: (0, 0)),
            pl.BlockSpec((1, Cout), lambda i: (0, 0)),
        ],
        out_specs=pl.BlockSpec((TR, Cout), lambda i: (i, 0)),
        out_shape=jax.ShapeDtypeStruct((M2, Cout), jnp.float32),
        compiler_params=pltpu.CompilerParams(
            dimension_semantics=("parallel",),
            vmem_limit_bytes=64 * 1024 * 1024,
        ),
    )(pooled2d, scale2, shift2)

    out = out2d.reshape(N, Hp, Wp, Cout)
    return jnp.transpose(out, (0, 3, 1, 2))


# speedup vs baseline: 1.4200x; 1.0144x over previous
import functools

import jax
import jax.numpy as jnp
from jax.experimental import pallas as pl
from jax.experimental.pallas import tpu as pltpu


def _conv_pool_stats_kernel(a_ref, t_ref, b_ref, w_ref, out_ref, stats_ref,
                            *, TH, W, C, nH):
    THp, Wp = TH // 2, W // 2
    h = pl.program_id(1)
    xm = a_ref[0]
    top = jnp.where(h == 0, 0.0, t_ref[0][:, 7:8, :])
    bot = jnp.where(h == nH - 1, 0.0, b_ref[0][:, 0:1, :])
    x3 = jnp.concatenate([top, xm, bot], axis=1)
    xt = jnp.transpose(x3.astype(jnp.bfloat16), (1, 2, 0))

    Cin = xt.shape[-1]
    z = jnp.zeros((TH + 2, 1, Cin), jnp.bfloat16)
    left = jnp.concatenate([z, xt[:, :-1, :]], axis=1)
    right = jnp.concatenate([xt[:, 1:, :], z], axis=1)
    xc = jnp.concatenate([left, xt, right], axis=-1)

    a = jnp.dot(xc.reshape((TH + 2) * W, 3 * Cin), w_ref[...],
                preferred_element_type=jnp.float32)
    a = a.reshape(TH + 2, W, 3 * C)

    conv = (a[0:TH, :, 0:C]
            + a[1:TH + 1, :, C:2 * C]
            + a[2:TH + 2, :, 2 * C:3 * C])

    c = conv.reshape(THp, 2, W, C)
    mh = jnp.maximum(c[:, 0], c[:, 1])
    mh2 = mh.reshape(THp, Wp, 2 * C)
    pooled = jnp.maximum(mh2[:, :, :C], mh2[:, :, C:])
    pooled = pooled.reshape(THp * Wp, C)
    out_ref[0] = pooled.astype(jnp.bfloat16)

    s = jnp.sum(pooled, axis=0, keepdims=True)
    ss = jnp.sum(pooled * pooled, axis=0, keepdims=True)
    stats_ref[0] = jnp.concatenate([s, ss], axis=0)


def _bn_relu_kernel(x_ref, scale_ref, shift_ref, o_ref):
    o_ref[...] = jnp.maximum(x_ref[...] * scale_ref[...] + shift_ref[...], 0.0)


def kernel(x_nchw, w, b, gamma, beta):
    del b
    eps = 1e-5
    N, Cin, H, W = x_nchw.shape
    Cout = w.shape[0]
    assert H % 2 == 0 and W % 2 == 0
    Hp, Wp = H // 2, W // 2

    TH = 32
    while H % TH != 0:
        TH //= 2
    assert TH % 8 == 0, "row tile must be a multiple of the 8-row halo blocks"
    THp = TH // 2
    nH = H // TH
    TH8 = TH // 8
    nR8 = H // 8

    wN = jnp.transpose(w, (3, 1, 2, 0)).reshape(3 * Cin, 3 * Cout)
    wN = wN.astype(jnp.bfloat16)

    cparams = pltpu.CompilerParams(
        dimension_semantics=("parallel", "parallel"),
        vmem_limit_bytes=100 * 1024 * 1024,
    )

    k1 = functools.partial(_conv_pool_stats_kernel, TH=TH, W=W, C=Cout, nH=nH)
    pooled, stats = pl.pallas_call(
        k1,
        grid=(N, nH),
        in_specs=[
            pl.BlockSpec((1, Cin, TH, W), lambda n, h: (n, 0, h, 0)),
            pl.BlockSpec((1, Cin, 8, W),
                         lambda n, h: (n, 0, jnp.maximum(h * TH8 - 1, 0), 0)),
            pl.BlockSpec((1, Cin, 8, W),
                         lambda n, h: (n, 0, jnp.minimum((h + 1) * TH8, nR8 - 1), 0)),
            pl.BlockSpec((3 * Cin, 3 * Cout), lambda n, h: (0, 0)),
        ],
        out_specs=[
            pl.BlockSpec((1, THp * Wp, Cout), lambda n, h: (n, h, 0)),
            pl.BlockSpec((1, 2, Cout), lambda n, h: (n * nH + h, 0, 0)),
        ],
        out_shape=[
            jax.ShapeDtypeStruct((N, Hp * Wp, Cout), jnp.bfloat16),
            jax.ShapeDtypeStruct((N * nH, 2, Cout), jnp.float32),
        ],
        compiler_params=cparams,
    )(x_nchw, x_nchw, x_nchw, wN)

    M2 = N * Hp * Wp
    ssum = jnp.sum(stats, axis=0)
    mean = ssum[0] / M2
    var = jnp.maximum(ssum[1] / M2 - mean * mean, 0.0)
    scale = gamma.astype(jnp.float32) * jax.lax.rsqrt(var + eps)
    shift = beta.astype(jnp.float32) - mean * scale
    scale2 = scale.reshape(1, Cout)
    shift2 = shift.reshape(1, Cout)

    TR = 4096
    while M2 % TR != 0:
        TR //= 2
    pooled2d = pooled.reshape(M2, Cout)
    out2d = pl.pallas_call(
        _bn_relu_kernel,
        grid=(M2 // TR,),
        in_specs=[
            pl.BlockSpec((TR, Cout), lambda i: (i, 0)),
            pl.BlockSpec((1, Cout), lambda i: (0, 0)),
            pl.BlockSpec((1, Cout), lambda i: (0, 0)),
        ],
        out_specs=pl.BlockSpec((TR, Cout), lambda i: (i, 0)),
        out_shape=jax.ShapeDtypeStruct((M2, Cout), jnp.float32),
        compiler_params=pltpu.CompilerParams(
            dimension_semantics=("parallel",),
            vmem_limit_bytes=64 * 1024 * 1024,
        ),
    )(pooled2d, scale2, shift2)

    out = out2d.reshape(N, Hp, Wp, Cout)
    return jnp.transpose(out, (0, 3, 1, 2))


# speedup vs baseline: 1.7951x; 1.2641x over previous
import functools

import jax
import jax.numpy as jnp
from jax.experimental import pallas as pl
from jax.experimental.pallas import tpu as pltpu


def _conv_pool_stats_kernel(a_ref, t_ref, b_ref, w_ref, out_ref, stats_ref,
                            *, TH, W, C, nH):
    THp, Wp = TH // 2, W // 2
    h = pl.program_id(1)
    xm = a_ref[0]
    top = jnp.where(h == 0, 0.0, t_ref[0][:, 7:8, :])
    bot = jnp.where(h == nH - 1, 0.0, b_ref[0][:, 0:1, :])
    x3 = jnp.concatenate([top, xm, bot], axis=1)
    xt = jnp.transpose(x3.astype(jnp.bfloat16), (1, 2, 0))

    Cin = xt.shape[-1]
    z = jnp.zeros((TH + 2, 1, Cin), jnp.bfloat16)
    left = jnp.concatenate([z, xt[:, :-1, :]], axis=1)
    right = jnp.concatenate([xt[:, 1:, :], z], axis=1)
    xc = jnp.concatenate([left, xt, right], axis=-1)

    a = jnp.dot(xc.reshape((TH + 2) * W, 3 * Cin), w_ref[...],
                preferred_element_type=jnp.float32)
    a = a.reshape(TH + 2, W, 3 * C)

    conv = (a[0:TH, :, 0:C]
            + a[1:TH + 1, :, C:2 * C]
            + a[2:TH + 2, :, 2 * C:3 * C])

    c = conv.reshape(THp, 2, W, C)
    mh = jnp.maximum(c[:, 0], c[:, 1])
    mh2 = mh.reshape(THp, Wp, 2 * C)
    pooled = jnp.maximum(mh2[:, :, :C], mh2[:, :, C:])
    pooled = pooled.reshape(THp * Wp, C)
    out_ref[0] = pooled.astype(jnp.bfloat16)

    s = jnp.sum(pooled, axis=0, keepdims=True)
    ss = jnp.sum(pooled * pooled, axis=0, keepdims=True)
    stats_ref[0] = jnp.concatenate([s, ss], axis=0)


def _bn_relu_kernel(x_ref, scale_ref, shift_ref, o_ref):
    o_ref[...] = jnp.maximum(x_ref[...] * scale_ref[...] + shift_ref[...], 0.0)


def kernel(x_nchw, w, b, gamma, beta):
    del b
    eps = 1e-5
    N, Cin, H, W = x_nchw.shape
    Cout = w.shape[0]
    assert H % 2 == 0 and W % 2 == 0
    Hp, Wp = H // 2, W // 2

    TH = 64
    while H % TH != 0:
        TH //= 2
    assert TH % 8 == 0, "row tile must be a multiple of the 8-row halo blocks"
    THp = TH // 2
    nH = H // TH
    TH8 = TH // 8
    nR8 = H // 8

    wN = jnp.transpose(w, (3, 1, 2, 0)).reshape(3 * Cin, 3 * Cout)
    wN = wN.astype(jnp.bfloat16)

    cparams = pltpu.CompilerParams(
        dimension_semantics=("parallel", "parallel"),
        vmem_limit_bytes=100 * 1024 * 1024,
    )

    k1 = functools.partial(_conv_pool_stats_kernel, TH=TH, W=W, C=Cout, nH=nH)
    pooled, stats = pl.pallas_call(
        k1,
        grid=(N, nH),
        in_specs=[
            pl.BlockSpec((1, Cin, TH, W), lambda n, h: (n, 0, h, 0)),
            pl.BlockSpec((1, Cin, 8, W),
                         lambda n, h: (n, 0, jnp.maximum(h * TH8 - 1, 0), 0)),
            pl.BlockSpec((1, Cin, 8, W),
                         lambda n, h: (n, 0, jnp.minimum((h + 1) * TH8, nR8 - 1), 0)),
            pl.BlockSpec((3 * Cin, 3 * Cout), lambda n, h: (0, 0)),
        ],
        out_specs=[
            pl.BlockSpec((1, THp * Wp, Cout), lambda n, h: (n, h, 0)),
            pl.BlockSpec((1, 2, Cout), lambda n, h: (n * nH + h, 0, 0)),
        ],
        out_shape=[
            jax.ShapeDtypeStruct((N, Hp * Wp, Cout), jnp.bfloat16),
            jax.ShapeDtypeStruct((N * nH, 2, Cout), jnp.float32),
        ],
        compiler_params=cparams,
    )(x_nchw, x_nchw, x_nchw, wN)

    M2 = N * Hp * Wp
    ssum = jnp.sum(stats, axis=0)
    mean = ssum[0] / M2
    var = jnp.maximum(ssum[1] / M2 - mean * mean, 0.0)
    scale = gamma.astype(jnp.float32) * jax.lax.rsqrt(var + eps)
    shift = beta.astype(jnp.float32) - mean * scale
    scale2 = scale.reshape(1, Cout)
    shift2 = shift.reshape(1, Cout)

    TR = 4096
    while M2 % TR != 0:
        TR //= 2
    pooled2d = pooled.reshape(M2, Cout)
    out2d = pl.pallas_call(
        _bn_relu_kernel,
        grid=(M2 // TR,),
        in_specs=[
            pl.BlockSpec((TR, Cout), lambda i: (i, 0)),
            pl.BlockSpec((1, Cout), lambda i: (0, 0)),
            pl.BlockSpec((1, Cout), lambda i: (0, 0)),
        ],
        out_specs=pl.BlockSpec((TR, Cout), lambda i: (i, 0)),
        out_shape=jax.ShapeDtypeStruct((M2, Cout), jnp.float32),
        compiler_params=pltpu.CompilerParams(
            dimension_semantics=("parallel",),
            vmem_limit_bytes=64 * 1024 * 1024,
        ),
    )(pooled2d, scale2, shift2)

    out = out2d.reshape(N, Hp, Wp, Cout)
    return jnp.transpose(out, (0, 3, 1, 2))


# speedup vs baseline: 2.0183x; 1.1243x over previous
import functools

import jax
import jax.numpy as jnp
from jax.experimental import pallas as pl
from jax.experimental.pallas import tpu as pltpu


def _conv_pool_stats_kernel(a_ref, t_ref, b_ref, w_ref, out_ref, stats_ref,
                            *, TH, W, C, nH):
    THp, Wp = TH // 2, W // 2
    h = pl.program_id(1)
    xm = a_ref[0]
    top = jnp.where(h == 0, 0.0, t_ref[0][:, 7:8, :])
    bot = jnp.where(h == nH - 1, 0.0, b_ref[0][:, 0:1, :])
    x3 = jnp.concatenate([top, xm, bot], axis=1)
    xt = jnp.transpose(x3.astype(jnp.bfloat16), (1, 2, 0))

    Cin = xt.shape[-1]
    z = jnp.zeros((TH + 2, 1, Cin), jnp.bfloat16)
    left = jnp.concatenate([z, xt[:, :-1, :]], axis=1)
    right = jnp.concatenate([xt[:, 1:, :], z], axis=1)
    xc = jnp.concatenate([left, xt, right], axis=-1)

    a = jnp.dot(xc.reshape((TH + 2) * W, 3 * Cin), w_ref[...],
                preferred_element_type=jnp.float32)
    a = a.reshape(TH + 2, W, 3 * C)

    conv = (a[0:TH, :, 0:C]
            + a[1:TH + 1, :, C:2 * C]
            + a[2:TH + 2, :, 2 * C:3 * C])

    c = conv.reshape(THp, 2, W, C)
    mh = jnp.maximum(c[:, 0], c[:, 1])
    mh2 = mh.reshape(THp, Wp, 2 * C)
    pooled = jnp.maximum(mh2[:, :, :C], mh2[:, :, C:])
    pooled = pooled.reshape(THp * Wp, C)
    out_ref[0] = pooled.astype(jnp.bfloat16)

    s = jnp.sum(pooled, axis=0, keepdims=True)
    ss = jnp.sum(pooled * pooled, axis=0, keepdims=True)
    stats_ref[0] = jnp.concatenate([s, ss], axis=0)


def _bn_relu_kernel(x_ref, scale_ref, shift_ref, o_ref):
    o_ref[...] = jnp.maximum(x_ref[...] * scale_ref[...] + shift_ref[...], 0.0)


def kernel(x_nchw, w, b, gamma, beta):
    del b
    eps = 1e-5
    N, Cin, H, W = x_nchw.shape
    Cout = w.shape[0]
    assert H % 2 == 0 and W % 2 == 0
    Hp, Wp = H // 2, W // 2

    TH = 128
    while H % TH != 0:
        TH //= 2
    assert TH % 8 == 0, "row tile must be a multiple of the 8-row halo blocks"
    THp = TH // 2
    nH = H // TH
    TH8 = TH // 8
    nR8 = H // 8

    wN = jnp.transpose(w, (3, 1, 2, 0)).reshape(3 * Cin, 3 * Cout)
    wN = wN.astype(jnp.bfloat16)

    cparams = pltpu.CompilerParams(
        dimension_semantics=("parallel", "parallel"),
        vmem_limit_bytes=100 * 1024 * 1024,
    )

    k1 = functools.partial(_conv_pool_stats_kernel, TH=TH, W=W, C=Cout, nH=nH)
    pooled, stats = pl.pallas_call(
        k1,
        grid=(N, nH),
        in_specs=[
            pl.BlockSpec((1, Cin, TH, W), lambda n, h: (n, 0, h, 0)),
            pl.BlockSpec((1, Cin, 8, W),
                         lambda n, h: (n, 0, jnp.maximum(h * TH8 - 1, 0), 0)),
            pl.BlockSpec((1, Cin, 8, W),
                         lambda n, h: (n, 0, jnp.minimum((h + 1) * TH8, nR8 - 1), 0)),
            pl.BlockSpec((3 * Cin, 3 * Cout), lambda n, h: (0, 0)),
        ],
        out_specs=[
            pl.BlockSpec((1, THp * Wp, Cout), lambda n, h: (n, h, 0)),
            pl.BlockSpec((1, 2, Cout), lambda n, h: (n * nH + h, 0, 0)),
        ],
        out_shape=[
            jax.ShapeDtypeStruct((N, Hp * Wp, Cout), jnp.bfloat16),
            jax.ShapeDtypeStruct((N * nH, 2, Cout), jnp.float32),
        ],
        compiler_params=cparams,
    )(x_nchw, x_nchw, x_nchw, wN)

    M2 = N * Hp * Wp
    ssum = jnp.sum(stats, axis=0)
    mean = ssum[0] / M2
    var = jnp.maximum(ssum[1] / M2 - mean * mean, 0.0)
    scale = gamma.astype(jnp.float32) * jax.lax.rsqrt(var + eps)
    shift = beta.astype(jnp.float32) - mean * scale
    scale2 = scale.reshape(1, Cout)
    shift2 = shift.reshape(1, Cout)

    TR = 4096
    while M2 % TR != 0:
        TR //= 2
    pooled2d = pooled.reshape(M2, Cout)
    out2d = pl.pallas_call(
        _bn_relu_kernel,
        grid=(M2 // TR,),
        in_specs=[
            pl.BlockSpec((TR, Cout), lambda i: (i, 0)),
            pl.BlockSpec((1, Cout), lambda i: (0, 0)),
            pl.BlockSpec((1, Cout), lambda i: (0, 0)),
        ],
        out_specs=pl.BlockSpec((TR, Cout), lambda i: (i, 0)),
        out_shape=jax.ShapeDtypeStruct((M2, Cout), jnp.float32),
        compiler_params=pltpu.CompilerParams(
            dimension_semantics=("parallel",),
            vmem_limit_bytes=64 * 1024 * 1024,
        ),
    )(pooled2d, scale2, shift2)

    out = out2d.reshape(N, Hp, Wp, Cout)
    return jnp.transpose(out, (0, 3, 1, 2))
